# Initial kernel scaffold; baseline (speedup 1.0000x reference)
import jax
import jax.numpy as jnp
from jax.experimental import pallas as pl


def kernel(alleles, positions, allele_table, kernel_table, bias_table):
    raise NotImplementedError("write your pallas kernel here")


# speedup vs baseline: 4.9263x; 4.9263x over previous
import functools

import jax
import jax.numpy as jnp
from jax import lax
from jax.experimental import pallas as pl
from jax.experimental.pallas import tpu as pltpu
from jax.experimental.pallas import tpu_sc as plsc

D = 16
NC = 2
NS = 16
NW = NC * NS
C = 128


def _build_kernel(n_total):
    assert n_total % NW == 0
    npw = n_total // NW
    assert npw % C == 0
    g = npw // C

    mesh = plsc.VectorSubcoreMesh(core_axis_name="c", subcore_axis_name="s")

    @functools.partial(
        pl.kernel,
        out_type=jax.ShapeDtypeStruct((n_total, D), jnp.float32),
        mesh=mesh,
        compiler_params=pltpu.CompilerParams(use_tc_tiling_on_sc=False),
        scratch_types=[
            pltpu.VMEM((C,), jnp.int32),
            pltpu.VMEM((C,), jnp.int32),
            pltpu.VMEM((C,), jnp.int32),
            pltpu.VMEM((2 * C, D), jnp.float32),
            pltpu.VMEM((C, D * D), jnp.float32),
            pltpu.VMEM((C, D), jnp.float32),
            pltpu.VMEM((C, D), jnp.float32),
            pltpu.SemaphoreType.DMA,
        ],
    )
    def emb_kernel(pos_hbm, al0_hbm, al1_hbm, atab_hbm, ktab_hbm, btab_hbm,
                   out_hbm, pos_v, al0_v, al1_v, arows_v, krows_v, brows_v,
                   outv, sem):
        wid = lax.axis_index("s") * NC + lax.axis_index("c")
        base0 = wid * npw

        def chunk_body(gi, carry):
            base = base0 + gi * C
            pltpu.sync_copy(pos_hbm.at[pl.ds(base, C)], pos_v)
            pltpu.sync_copy(al0_hbm.at[pl.ds(base, C)], al0_v)
            pltpu.sync_copy(al1_hbm.at[pl.ds(base, C)], al1_v)
            pltpu.async_copy(ktab_hbm.at[pos_v], krows_v, sem).wait()
            pltpu.async_copy(btab_hbm.at[pos_v], brows_v, sem).wait()
            pltpu.async_copy(atab_hbm.at[al0_v], arows_v.at[pl.ds(0, C)],
                             sem).wait()
            pltpu.async_copy(atab_hbm.at[al1_v], arows_v.at[pl.ds(C, C)],
                             sem).wait()

            dnums = lax.GatherDimensionNumbers(
                offset_dims=(), collapsed_slice_dims=(0,),
                start_index_map=(0,))

            def pos_body(c, carry2):
                a = arows_v[c, :] + arows_v[C + c, :]
                acc = brows_v[c, :]
                for d1 in range(D):
                    bc = lax.gather(
                        a, jnp.full((16, 1), d1, jnp.int32), dnums,
                        slice_sizes=(1,),
                        mode=lax.GatherScatterMode.PROMISE_IN_BOUNDS)
                    acc = acc + bc * krows_v[c, pl.ds(d1 * D, D)]
                outv[c, :] = acc
                return carry2

            lax.fori_loop(0, C, pos_body, 0, unroll=2)
            pltpu.sync_copy(outv, out_hbm.at[pl.ds(base, C)])
            return carry

        lax.fori_loop(0, g, chunk_body, 0)

    return emb_kernel


def kernel(alleles, positions, allele_table, kernel_table, bias_table):
    b, p, _ = alleles.shape
    n = b * p
    pos_flat = positions.reshape(n)
    al0 = alleles[:, :, 0].reshape(n)
    al1 = alleles[:, :, 1].reshape(n)
    out = _build_kernel(n)(pos_flat, al0, al1, allele_table, kernel_table,
                           bias_table)
    return out.reshape(b, p, D)


# speedup vs baseline: 7.7705x; 1.5774x over previous
import functools

import jax
import jax.numpy as jnp
from jax import lax
from jax.experimental import pallas as pl
from jax.experimental.pallas import tpu as pltpu
from jax.experimental.pallas import tpu_sc as plsc

D = 16
NC = 2
NS = 16
NW = NC * NS
C = 128

_BCAST_DNUMS = lax.GatherDimensionNumbers(
    offset_dims=(), collapsed_slice_dims=(0,), start_index_map=(0,))


def _build_kernel(n_total):
    assert n_total % (NW * C) == 0
    npw = n_total // NW
    g_cnt = npw // C
    assert g_cnt % 2 == 0
    g_half = g_cnt // 2

    mesh = plsc.VectorSubcoreMesh(core_axis_name="c", subcore_axis_name="s")

    @functools.partial(
        pl.kernel,
        out_type=jax.ShapeDtypeStruct((n_total, D), jnp.float32),
        mesh=mesh,
        compiler_params=pltpu.CompilerParams(use_tc_tiling_on_sc=False),
        scratch_types=[
            pltpu.VMEM((2, 3, C), jnp.int32),
            pltpu.VMEM((2, 2 * C, D), jnp.float32),
            pltpu.VMEM((2, C, D * D), jnp.float32),
            pltpu.VMEM((2, C, D), jnp.float32),
            pltpu.VMEM((2, C, D), jnp.float32),
            pltpu.SemaphoreType.DMA,
            pltpu.SemaphoreType.DMA,
            pltpu.SemaphoreType.DMA,
            pltpu.SemaphoreType.DMA,
            pltpu.SemaphoreType.DMA,
            pltpu.SemaphoreType.DMA,
        ],
    )
    def emb_kernel(idx_hbm, atab_hbm, ktab_hbm, btab_hbm, out_hbm,
                   idx_v, arows_v, krows_v, brows_v, outv,
                   sem_i0, sem_i1, sem_g0, sem_g1, sem_o0, sem_o1):
        wid = lax.axis_index("s") * NC + lax.axis_index("c")
        base0 = wid * npw
        sem_i = (sem_i0, sem_i1)
        sem_g = (sem_g0, sem_g1)
        sem_o = (sem_o0, sem_o1)

        def idx_copy(slot, base):
            pltpu.make_async_copy(idx_hbm.at[:, pl.ds(base, C)],
                                  idx_v.at[slot], sem_i[slot]).start()
            pltpu.make_async_copy(idx_hbm.at[:, pl.ds(base, C)],
                                  idx_v.at[slot], sem_i[slot]).wait()

        def gather_descs(slot):
            return (
                pltpu.make_async_copy(ktab_hbm.at[idx_v.at[slot, 0]],
                                      krows_v.at[slot], sem_g[slot]),
                pltpu.make_async_copy(btab_hbm.at[idx_v.at[slot, 0]],
                                      brows_v.at[slot], sem_g[slot]),
                pltpu.make_async_copy(atab_hbm.at[idx_v.at[slot, 1]],
                                      arows_v.at[slot, pl.ds(0, C)],
                                      sem_g[slot]),
                pltpu.make_async_copy(atab_hbm.at[idx_v.at[slot, 2]],
                                      arows_v.at[slot, pl.ds(C, C)],
                                      sem_g[slot]),
            )

        def gathers_start(slot):
            for d in gather_descs(slot):
                d.start()

        def gathers_wait(slot):
            for d in gather_descs(slot):
                d.wait()

        def out_desc(slot, base):
            return pltpu.make_async_copy(outv.at[slot],
                                         out_hbm.at[pl.ds(base, C)],
                                         sem_o[slot])

        def prefetch(slot, base):
            idx_copy(slot, base)
            gathers_start(slot)

        def compute(slot):
            def pos_body(c, carry):
                a = arows_v[slot, c, :] + arows_v[slot, C + c, :]
                acc = brows_v[slot, c, :]
                for d1 in range(D):
                    bc = lax.gather(
                        a, jnp.full((16, 1), d1, jnp.int32), _BCAST_DNUMS,
                        slice_sizes=(1,),
                        mode=lax.GatherScatterMode.PROMISE_IN_BOUNDS)
                    acc = acc + bc * krows_v[slot, c, pl.ds(d1 * D, D)]
                outv[slot, c, :] = acc
                return carry

            lax.fori_loop(0, C, pos_body, 0, unroll=2)

        prefetch(0, base0)

        def body(gg, carry):
            b0 = base0 + (2 * gg) * C
            b1 = b0 + C

            prefetch(1, b1)

            gathers_wait(0)

            @pl.when(gg > 0)
            def _():
                out_desc(0, b0).wait()

            compute(0)
            out_desc(0, b0).start()

            @pl.when(gg + 1 < g_half)
            def _():
                prefetch(0, b0 + 2 * C)

            gathers_wait(1)

            @pl.when(gg > 0)
            def _():
                out_desc(1, b1).wait()

            compute(1)
            out_desc(1, b1).start()
            return carry

        lax.fori_loop(0, g_half, body, 0)
        last = base0 + (g_cnt - 2) * C
        out_desc(0, last).wait()
        out_desc(1, last + C).wait()

    return emb_kernel


def kernel(alleles, positions, allele_table, kernel_table, bias_table):
    b, p, _ = alleles.shape
    n = b * p
    idx_all = jnp.stack([
        positions.reshape(n),
        alleles[:, :, 0].reshape(n),
        alleles[:, :, 1].reshape(n),
    ])
    out = _build_kernel(n)(idx_all, allele_table, kernel_table, bias_table)
    return out.reshape(b, p, D)


# speedup vs baseline: 9.7852x; 1.2593x over previous
import functools

import jax
import jax.numpy as jnp
from jax import lax
from jax.experimental import pallas as pl
from jax.experimental.pallas import tpu as pltpu
from jax.experimental.pallas import tpu_sc as plsc

D = 16
NC = 2
NS = 16
NW = NC * NS
C = 128
CR = C // 8

_BCAST_DNUMS = lax.GatherDimensionNumbers(
    offset_dims=(), collapsed_slice_dims=(0,), start_index_map=(0,))

_MESH = plsc.VectorSubcoreMesh(core_axis_name="c", subcore_axis_name="s")


def _wid():
    return lax.axis_index("s") * NC + lax.axis_index("c")


def _pack_slices(c):
    return c >> 3, pl.ds((c & 7) * D, D)


def _build_pack_kernel(n_total):
    npw = n_total // NW
    g_cnt = npw // C
    g_half = g_cnt // 2

    @functools.partial(
        pl.kernel,
        out_type=(jax.ShapeDtypeStruct((n_total // 8, 128), jnp.float32),
                  jax.ShapeDtypeStruct((n_total // 8, 128), jnp.float32)),
        mesh=_MESH,
        compiler_params=pltpu.CompilerParams(use_tc_tiling_on_sc=False),
        scratch_types=[
            pltpu.VMEM((2, 3, C), jnp.int32),
            pltpu.VMEM((2, 2 * C, D), jnp.float32),
            pltpu.VMEM((2, C, D), jnp.float32),
            pltpu.VMEM((2, CR, 128), jnp.float32),
            pltpu.VMEM((2, CR, 128), jnp.float32),
            pltpu.SemaphoreType.DMA,
            pltpu.SemaphoreType.DMA,
            pltpu.SemaphoreType.DMA,
            pltpu.SemaphoreType.DMA,
            pltpu.SemaphoreType.DMA,
            pltpu.SemaphoreType.DMA,
        ],
    )
    def pack_kernel(pos_hbm, al0_hbm, al1_hbm, atab_hbm, btab_hbm,
                    asum_hbm, bpack_hbm,
                    idx_v, arows_v, brows_v, apack_v, bpack_v,
                    sem_i0, sem_i1, sem_g0, sem_g1, sem_o0, sem_o1):
        base0 = _wid() * npw
        sem_i = (sem_i0, sem_i1)
        sem_g = (sem_g0, sem_g1)
        sem_o = (sem_o0, sem_o1)

        def idx_descs(slot, base):
            return (
                pltpu.make_async_copy(pos_hbm.at[pl.ds(base, C)],
                                      idx_v.at[slot, 0], sem_i[slot]),
                pltpu.make_async_copy(al0_hbm.at[pl.ds(base, C)],
                                      idx_v.at[slot, 1], sem_i[slot]),
                pltpu.make_async_copy(al1_hbm.at[pl.ds(base, C)],
                                      idx_v.at[slot, 2], sem_i[slot]),
            )

        def gather_descs(slot):
            return (
                pltpu.make_async_copy(btab_hbm.at[idx_v.at[slot, 0]],
                                      brows_v.at[slot], sem_g[slot]),
                pltpu.make_async_copy(atab_hbm.at[idx_v.at[slot, 1]],
                                      arows_v.at[slot, pl.ds(0, C)],
                                      sem_g[slot]),
                pltpu.make_async_copy(atab_hbm.at[idx_v.at[slot, 2]],
                                      arows_v.at[slot, pl.ds(C, C)],
                                      sem_g[slot]),
            )

        def out_descs(slot, base):
            return (
                pltpu.make_async_copy(apack_v.at[slot],
                                      asum_hbm.at[pl.ds(base >> 3, CR)],
                                      sem_o[slot]),
                pltpu.make_async_copy(bpack_v.at[slot],
                                      bpack_hbm.at[pl.ds(base >> 3, CR)],
                                      sem_o[slot]),
            )

        def prefetch(slot, base):
            for d in idx_descs(slot, base):
                d.start()
            for d in idx_descs(slot, base):
                d.wait()
            for d in gather_descs(slot):
                d.start()

        def compute(slot):
            @plsc.parallel_loop(0, C, 1, unroll=4)
            def _(c):
                cb, cs = _pack_slices(c)
                apack_v[slot, cb, cs] = (arows_v[slot, c, :] +
                                         arows_v[slot, C + c, :])
                bpack_v[slot, cb, cs] = brows_v[slot, c, :]

        prefetch(0, base0)

        def body(gg, carry):
            b0 = base0 + (2 * gg) * C
            b1 = b0 + C

            prefetch(1, b1)
            for d in gather_descs(0):
                d.wait()

            @pl.when(gg > 0)
            def _():
                for d in out_descs(0, b0):
                    d.wait()

            compute(0)
            for d in out_descs(0, b0):
                d.start()

            @pl.when(gg + 1 < g_half)
            def _():
                prefetch(0, b0 + 2 * C)

            for d in gather_descs(1):
                d.wait()

            @pl.when(gg > 0)
            def _():
                for d in out_descs(1, b1):
                    d.wait()

            compute(1)
            for d in out_descs(1, b1):
                d.start()
            return carry

        lax.fori_loop(0, g_half, body, 0)
        last = base0 + (g_cnt - 2) * C
        for d in out_descs(0, last):
            d.wait()
        for d in out_descs(1, last + C):
            d.wait()

    return pack_kernel


def _build_matvec_kernel(n_total):
    npw = n_total // NW
    g_cnt = npw // C
    g_half = g_cnt // 2

    @functools.partial(
        pl.kernel,
        out_type=jax.ShapeDtypeStruct((n_total // 8, 128), jnp.float32),
        mesh=_MESH,
        scratch_types=[
            pltpu.VMEM((2, C), jnp.int32),
            pltpu.VMEM((2, C, D * D), jnp.float32),
            pltpu.VMEM((2, CR, 128), jnp.float32),
            pltpu.VMEM((2, CR, 128), jnp.float32),
            pltpu.VMEM((2, CR, 128), jnp.float32),
            pltpu.SemaphoreType.DMA,
            pltpu.SemaphoreType.DMA,
            pltpu.SemaphoreType.DMA,
            pltpu.SemaphoreType.DMA,
            pltpu.SemaphoreType.DMA,
            pltpu.SemaphoreType.DMA,
        ],
    )
    def matvec_kernel(pos_hbm, ktab_hbm, asum_hbm, bpack_hbm, out_hbm,
                      pos_v, krows_v, apack_v, bpack_v, outv,
                      sem_i0, sem_i1, sem_g0, sem_g1, sem_o0, sem_o1):
        base0 = _wid() * npw

        def _rows(base):
            return pl.ds(pl.multiple_of(base >> 3, 8), CR)
        sem_i = (sem_i0, sem_i1)
        sem_g = (sem_g0, sem_g1)
        sem_o = (sem_o0, sem_o1)

        def idx_desc(slot, base):
            return pltpu.make_async_copy(pos_hbm.at[pl.ds(base, C)],
                                         pos_v.at[slot], sem_i[slot])

        def lin_descs(slot, base):
            return (
                pltpu.make_async_copy(asum_hbm.at[_rows(base)],
                                      apack_v.at[slot], sem_g[slot]),
                pltpu.make_async_copy(bpack_hbm.at[_rows(base)],
                                      bpack_v.at[slot], sem_g[slot]),
            )

        def ktab_desc(slot):
            return pltpu.make_async_copy(ktab_hbm.at[pos_v.at[slot]],
                                         krows_v.at[slot], sem_g[slot])

        def out_desc(slot, base):
            return pltpu.make_async_copy(outv.at[slot],
                                         out_hbm.at[_rows(base)],
                                         sem_o[slot])

        def prefetch(slot, base):
            for d in lin_descs(slot, base):
                d.start()
            idx_desc(slot, base).start()
            idx_desc(slot, base).wait()
            ktab_desc(slot).start()

        def gathers_wait(slot, base):
            ktab_desc(slot).wait()
            for d in lin_descs(slot, base):
                d.wait()

        def compute(slot):
            @plsc.parallel_loop(0, C, 1, unroll=2)
            def _(c):
                cb, cs = _pack_slices(c)
                a = apack_v[slot, cb, cs]
                acc = bpack_v[slot, cb, cs]
                for d1 in range(D):
                    bc = lax.gather(
                        a, jnp.full((16, 1), d1, jnp.int32), _BCAST_DNUMS,
                        slice_sizes=(1,),
                        mode=lax.GatherScatterMode.PROMISE_IN_BOUNDS)
                    acc = acc + bc * krows_v[slot, c, pl.ds(d1 * D, D)]
                outv[slot, cb, cs] = acc

        prefetch(0, base0)

        def body(gg, carry):
            b0 = base0 + (2 * gg) * C
            b1 = b0 + C

            prefetch(1, b1)
            gathers_wait(0, b0)

            @pl.when(gg > 0)
            def _():
                out_desc(0, b0).wait()

            compute(0)
            out_desc(0, b0).start()

            @pl.when(gg + 1 < g_half)
            def _():
                prefetch(0, b0 + 2 * C)

            gathers_wait(1, b1)

            @pl.when(gg > 0)
            def _():
                out_desc(1, b1).wait()

            compute(1)
            out_desc(1, b1).start()
            return carry

        lax.fori_loop(0, g_half, body, 0)
        last = base0 + (g_cnt - 2) * C
        out_desc(0, last).wait()
        out_desc(1, last + C).wait()

    return matvec_kernel


def kernel(alleles, positions, allele_table, kernel_table, bias_table):
    b, p, _ = alleles.shape
    n = b * p
    pos_flat = positions.reshape(n)
    al0 = alleles[:, :, 0].reshape(n)
    al1 = alleles[:, :, 1].reshape(n)
    asum_p, bias_p = _build_pack_kernel(n)(pos_flat, al0, al1, allele_table,
                                           bias_table)
    out_p = _build_matvec_kernel(n)(pos_flat, kernel_table, asum_p, bias_p)
    return out_p.reshape(b, p, D)
